# Initial kernel scaffold; baseline (speedup 1.0000x reference)
import functools

import jax
import jax.numpy as jnp
from jax import lax
from jax.experimental import pallas as pl
from jax.experimental.pallas import tpu as pltpu
from jax.experimental.pallas import tpu_sc as plsc

HID = 128
N = 10000
E = 320000
P = 320000
NCLS = 10
NG = 64
VE = 100
EPS = 1e-5
G4 = 4 * HID
SRCW = G4 + HID
NCORES = 2
NSUB = 16
NW = NCORES * NSUB
NPAD = 10240
VROWS = NPAD // NW
PT = P // NW
K = 16
NCHUNK = PT // K
AK = 80
ANB = 5
ANCH = PT // AK // ANB
NTROW = N // NSUB
TAROWS = 104

_mesh = plsc.VectorSubcoreMesh(core_axis_name="c", subcore_axis_name="s")


def _sig(v):
    return 1.0 / (1.0 + jnp.exp(-v))


def _th(v):
    return 2.0 / (1.0 + jnp.exp(-2.0 * v)) - 1.0


@functools.partial(
    pl.kernel,
    out_type=jax.ShapeDtypeStruct((NPAD, HID), jnp.float32),
    mesh=_mesh,
    scratch_types=[
        pltpu.VMEM((VROWS,), jnp.int32),
        pltpu.VMEM((VROWS, HID), jnp.float32),
        pltpu.SemaphoreType.DMA,
    ],
)
def _gather_w(xpad_hbm, embv_hbm, w_hbm, idx_v, rows_v, sem):
    wid = lax.axis_index("s") * NCORES + lax.axis_index("c")
    base = wid * VROWS
    pltpu.sync_copy(xpad_hbm.at[pl.ds(base, VROWS)], idx_v)
    cps = [
        pltpu.async_copy(
            embv_hbm.at[idx_v.at[pl.ds(i * AK, AK)]],
            rows_v.at[pl.ds(i * AK, AK)],
            sem,
        )
        for i in range(VROWS // AK)
    ]
    for cp in cps:
        cp.wait()
    pltpu.sync_copy(rows_v, w_hbm.at[pl.ds(base, VROWS)])


@functools.partial(
    pl.kernel,
    out_type=jax.ShapeDtypeStruct((NCORES, N, HID), jnp.float32),
    mesh=_mesh,
    scratch_types=[
        pltpu.VMEM((PT,), jnp.int32),
        pltpu.VMEM((NCHUNK, K), jnp.int32),
        pltpu.VMEM((PT,), jnp.int32),
        pltpu.VMEM((PT,), jnp.int32),
        pltpu.VMEM((2, K, G4), jnp.float32),
        pltpu.VMEM((2, K, SRCW), jnp.float32),
        pltpu.VMEM((2, K, G4), jnp.float32),
        pltpu.VMEM((K, HID), jnp.float32),
        pltpu.VMEM_SHARED((N, HID), jnp.float32),
        pltpu.SemaphoreType.DMA,
        pltpu.SemaphoreType.DMA,
        pltpu.SemaphoreType.DMA,
    ],
)
def _paths(tdst_hbm, tsrc_hbm, tattr_hbm, srcs_hbm, dsts_hbm, es_hbm,
           eattr_hbm, zeros_hbm, out_hbm,
           src_loc, dst_loc, e_loc, attr_loc, bufA, bufS, bufB, h_out, acc,
           sem0, sem1, semp):
    sid = lax.axis_index("s")
    cid = lax.axis_index("c")
    wid = sid * NCORES + cid
    sems = (sem0, sem1)

    pltpu.sync_copy(srcs_hbm.at[wid], src_loc)
    pltpu.sync_copy(dsts_hbm.at[wid], dst_loc)
    pltpu.sync_copy(es_hbm.at[wid], e_loc)
    pltpu.sync_copy(zeros_hbm, acc.at[pl.ds(sid * NTROW, NTROW)])

    def abatch(t, carry):
        cps = [
            pltpu.async_copy(
                eattr_hbm.at[e_loc.at[pl.ds((t * ANB + u) * AK, AK)]],
                attr_loc.at[pl.ds((t * ANB + u) * AK, AK)],
                semp,
            )
            for u in range(ANB)
        ]
        for cp in cps:
            cp.wait()
        return carry

    lax.fori_loop(0, ANCH, abatch, 0)
    plsc.subcore_barrier()

    def start(g, b):
        pltpu.async_copy(tdst_hbm.at[dst_loc.at[g]], bufA.at[b], sems[b])
        pltpu.async_copy(tsrc_hbm.at[src_loc.at[pl.ds(g * K, K)]],
                         bufS.at[b], sems[b])
        pltpu.async_copy(tattr_hbm.at[attr_loc.at[pl.ds(g * K, K)]],
                         bufB.at[b], sems[b])

    def drain(b):
        pltpu.make_async_copy(tdst_hbm.at[pl.ds(0, K)], bufA.at[b],
                              sems[b]).wait()
        pltpu.make_async_copy(tsrc_hbm.at[pl.ds(0, K)], bufS.at[b],
                              sems[b]).wait()
        pltpu.make_async_copy(tattr_hbm.at[pl.ds(0, K)], bufB.at[b],
                              sems[b]).wait()

    def compute(g, b):
        def pbody(j, carry):
            for q in range(HID // 16):
                qo = q * 16
                gi = (bufA[b, j, pl.ds(qo, 16)]
                      + bufS[b, j, pl.ds(qo, 16)]
                      + bufB[b, j, pl.ds(qo, 16)])
                gf = (bufA[b, j, pl.ds(HID + qo, 16)]
                      + bufS[b, j, pl.ds(HID + qo, 16)]
                      + bufB[b, j, pl.ds(HID + qo, 16)])
                gg = (bufA[b, j, pl.ds(2 * HID + qo, 16)]
                      + bufS[b, j, pl.ds(2 * HID + qo, 16)]
                      + bufB[b, j, pl.ds(2 * HID + qo, 16)])
                go = (bufA[b, j, pl.ds(3 * HID + qo, 16)]
                      + bufS[b, j, pl.ds(3 * HID + qo, 16)]
                      + bufB[b, j, pl.ds(3 * HID + qo, 16)])
                c0q = bufS[b, j, pl.ds(G4 + qo, 16)]
                c1 = _sig(gf) * c0q + _sig(gi) * _th(gg)
                h_out[j, pl.ds(qo, 16)] = _sig(go) * _th(c1)
            return carry

        lax.fori_loop(0, K, pbody, 0)
        pltpu.sync_copy(h_out, acc.at[dst_loc.at[g]], add=True)

    start(0, 0)

    def pair(t, carry):
        g0 = 2 * t
        start(g0 + 1, 1)
        drain(0)
        compute(g0, 0)
        start(g0 + 2, 0)
        drain(1)
        compute(g0 + 1, 1)
        return carry

    lax.fori_loop(0, (NCHUNK - 1) // 2, pair, 0)
    drain(0)
    compute(NCHUNK - 1, 0)

    plsc.subcore_barrier()
    pltpu.sync_copy(acc.at[pl.ds(sid * NTROW, NTROW)],
                    out_hbm.at[cid, pl.ds(sid * NTROW, NTROW)])


def _tables_tc(wpad, wihxT, whhT, bias2):
    BR = 1024

    def body(w_ref, wx_ref, wh_ref, b_ref, tdst_ref, tsrc_ref):
        a = jnp.dot(w_ref[...], wx_ref[...],
                    preferred_element_type=jnp.float32)
        tdst_ref[...] = a
        g0 = a + b_ref[...]
        c0 = _sig(g0[:, :HID]) * jnp.tanh(g0[:, 2 * HID:3 * HID])
        h0 = _sig(g0[:, 3 * HID:]) * jnp.tanh(c0)
        cc = jnp.dot(h0, wh_ref[...], preferred_element_type=jnp.float32)
        tsrc_ref[:, :G4] = cc
        tsrc_ref[:, G4:] = c0

    return pl.pallas_call(
        body,
        grid=(NPAD // BR,),
        in_specs=[
            pl.BlockSpec((BR, HID), lambda i: (i, 0)),
            pl.BlockSpec((HID, G4), lambda i: (0, 0)),
            pl.BlockSpec((HID, G4), lambda i: (0, 0)),
            pl.BlockSpec((1, G4), lambda i: (0, 0)),
        ],
        out_specs=[
            pl.BlockSpec((BR, G4), lambda i: (i, 0)),
            pl.BlockSpec((BR, SRCW), lambda i: (i, 0)),
        ],
        out_shape=[
            jax.ShapeDtypeStruct((NPAD, G4), jnp.float32),
            jax.ShapeDtypeStruct((NPAD, SRCW), jnp.float32),
        ],
    )(wpad, wihxT, whhT, bias2)


def _tattr_tc(embe_pad, wiheT, bias2):
    def body(e_ref, wx_ref, b_ref, o_ref):
        o_ref[...] = jnp.dot(e_ref[...], wx_ref[...],
                             preferred_element_type=jnp.float32) + b_ref[...]

    return pl.pallas_call(
        body,
        out_shape=jax.ShapeDtypeStruct((TAROWS, G4), jnp.float32),
    )(embe_pad, wiheT, bias2)


def _post_tc(part2, batch2, bng, bnb, w1T, b1, g1, e1, w2T, b2, g2, e2,
             l1T, lb1, l2Tp, lb2p):
    def bn(h, g, bb):
        m = jnp.mean(h, axis=0, keepdims=True)
        v = jnp.mean((h - m) ** 2, axis=0, keepdims=True)
        return (h - m) / jnp.sqrt(v + EPS) * g + bb

    def body(part_ref, batch_ref, bng_ref, bnb_ref, w1_ref, b1_ref, g1_ref,
             e1_ref, w2_ref, b2_ref, g2_ref, e2_ref, l1_ref, lb1_ref,
             l2_ref, lb2_ref, out_ref):
        h = part_ref[:N, :] + part_ref[N:, :]
        h = bn(h, bng_ref[...], bnb_ref[...])
        h = jnp.dot(h, w1_ref[...], preferred_element_type=jnp.float32)
        h = jnp.maximum(bn(h + b1_ref[...], g1_ref[...], e1_ref[...]), 0.0)
        h = jnp.dot(h, w2_ref[...], preferred_element_type=jnp.float32)
        h = jnp.maximum(bn(h + b2_ref[...], g2_ref[...], e2_ref[...]), 0.0)
        seg = lax.broadcasted_iota(jnp.int32, (NG, N), 0)
        oh = (seg == batch_ref[...]).astype(jnp.float32)
        pooled = jnp.dot(oh, h, preferred_element_type=jnp.float32)
        o1 = jnp.maximum(
            jnp.dot(pooled, l1_ref[...],
                    preferred_element_type=jnp.float32) + lb1_ref[...], 0.0)
        out_ref[...] = jnp.dot(
            o1, l2_ref[...], preferred_element_type=jnp.float32) + lb2_ref[...]

    return pl.pallas_call(
        body,
        out_shape=jax.ShapeDtypeStruct((NG, HID), jnp.float32),
    )(part2, batch2, bng, bnb, w1T, b1, g1, e1, w2T, b2, g2, e2,
      l1T, lb1, l2Tp, lb2p)


def kernel(x, edge_attr, path_2, edge_indices_2, batch, emb_v, emb_e,
           W_ih, W_hh, b_ih, b_hh, bn_g, bn_b, mlp_W1, mlp_b1, bn1_g, bn1_b,
           mlp_W2, mlp_b2, bn2_g, bn2_b, lin1_W, lin1_b, lin2_W, lin2_b):
    f32 = jnp.float32
    x_pad = jnp.pad(x, (0, NPAD - N))
    srcs = path_2[:, 0].reshape(NW, PT)
    dsts = path_2[:, 1].reshape(NW, NCHUNK, K)
    es = edge_indices_2[:, 0].reshape(NW, PT)
    wihxT = W_ih[:, :HID].T.astype(f32)
    wiheT = W_ih[:, HID:].T.astype(f32)
    whhT = W_hh.T.astype(f32)
    bias2 = (b_ih + b_hh).reshape(1, G4)
    embe_pad = jnp.pad(emb_e, ((0, TAROWS - VE), (0, 0)))
    zeros = jnp.zeros((NTROW, HID), f32)

    wpad = _gather_w(x_pad, emb_v)
    tdst, tsrc = _tables_tc(wpad, wihxT, whhT, bias2)
    tattr = _tattr_tc(embe_pad, wiheT, bias2)
    part = _paths(tdst, tsrc, tattr, srcs, dsts, es, edge_attr, zeros)
    out = _post_tc(
        part.reshape(NCORES * N, HID), batch.reshape(1, N),
        bn_g.reshape(1, HID), bn_b.reshape(1, HID),
        mlp_W1.T, mlp_b1.reshape(1, HID),
        bn1_g.reshape(1, HID), bn1_b.reshape(1, HID),
        mlp_W2.T, mlp_b2.reshape(1, HID),
        bn2_g.reshape(1, HID), bn2_b.reshape(1, HID),
        lin1_W.T, lin1_b.reshape(1, HID),
        jnp.pad(lin2_W.T, ((0, 0), (0, HID - NCLS))),
        jnp.pad(lin2_b, (0, HID - NCLS)).reshape(1, HID),
    )
    return out[:, :NCLS]


# speedup vs baseline: 3.9747x; 3.9747x over previous
import functools

import jax
import jax.numpy as jnp
from jax import lax
from jax.experimental import pallas as pl
from jax.experimental.pallas import tpu as pltpu
from jax.experimental.pallas import tpu_sc as plsc

HID = 128
N = 10000
E = 320000
P = 320000
NCLS = 10
NG = 64
VE = 100
EPS = 1e-5
G4 = 4 * HID
SRCW = G4 + HID
NCORES = 2
NSUB = 16
NW = NCORES * NSUB
NPAD = 10240
VROWS = NPAD // NW
PT = P // NW
K = 8
NCHUNK = PT // K
AK = 80
ACH = 400
ANB = ACH // AK
ANIT = PT // ACH
NTROW = 624
NTAIL = N - NTROW * NSUB
TAROWS = 104
SMASK = (1 << 14) - 1

_mesh = plsc.VectorSubcoreMesh(core_axis_name="c", subcore_axis_name="s")


def _sig(v):
    return 1.0 / (1.0 + jnp.exp(-v))


def _th(v):
    return 2.0 / (1.0 + jnp.exp(-2.0 * v)) - 1.0


@functools.partial(
    pl.kernel,
    out_type=(
        jax.ShapeDtypeStruct((NPAD, HID), jnp.float32),
        jax.ShapeDtypeStruct((P,), jnp.int32),
    ),
    mesh=_mesh,
    scratch_types=[
        pltpu.VMEM((VROWS,), jnp.int32),
        pltpu.VMEM((VROWS, HID), jnp.float32),
        pltpu.VMEM((2 * ACH,), jnp.int32),
        pltpu.VMEM((2 * ACH,), jnp.int32),
        pltpu.VMEM((ACH,), jnp.int32),
        pltpu.VMEM((ACH,), jnp.int32),
        pltpu.SemaphoreType.DMA,
        pltpu.SemaphoreType.DMA,
        pltpu.SemaphoreType.DMA,
    ],
)
def _gather_stage(xpad_hbm, embv_hbm, es_hbm, srcs_hbm, eattr_hbm,
                  w_hbm, packed_hbm,
                  idx_v, rows_v, ebuf, sbuf, abuf, pbuf, semw, seml, semg):
    wid = lax.axis_index("s") * NCORES + lax.axis_index("c")

    base = wid * VROWS
    pltpu.sync_copy(xpad_hbm.at[pl.ds(base, VROWS)], idx_v)
    cps = [
        pltpu.async_copy(
            embv_hbm.at[idx_v.at[pl.ds(i * AK, AK)]],
            rows_v.at[pl.ds(i * AK, AK)],
            semw,
        )
        for i in range(VROWS // AK)
    ]
    for cp in cps:
        cp.wait()
    pltpu.sync_copy(rows_v, w_hbm.at[pl.ds(base, VROWS)])

    pbase = wid * PT

    def loads(i, b):
        pltpu.async_copy(es_hbm.at[pl.ds(pbase + i * ACH, ACH)],
                         ebuf.at[pl.ds(b * ACH, ACH)], seml)
        pltpu.async_copy(srcs_hbm.at[pl.ds(pbase + i * ACH, ACH)],
                         sbuf.at[pl.ds(b * ACH, ACH)], seml)

    def ldrain(b):
        pltpu.make_async_copy(es_hbm.at[pl.ds(0, ACH)],
                              ebuf.at[pl.ds(b * ACH, ACH)], seml).wait()
        pltpu.make_async_copy(es_hbm.at[pl.ds(0, ACH)],
                              sbuf.at[pl.ds(b * ACH, ACH)], seml).wait()

    def step(i, b):
        ldrain(b)
        gcps = [
            pltpu.async_copy(
                eattr_hbm.at[ebuf.at[pl.ds(b * ACH + u * AK, AK)]],
                abuf.at[pl.ds(u * AK, AK)],
                semg,
            )
            for u in range(ANB)
        ]
        for cp in gcps:
            cp.wait()
        for v in range(ACH // 16):
            sl = pl.ds(v * 16, 16)
            pbuf[sl] = jnp.bitwise_or(
                jnp.left_shift(abuf[sl], 14),
                sbuf[pl.ds(b * ACH + v * 16, 16)])
        pltpu.sync_copy(pbuf, packed_hbm.at[pl.ds(pbase + i * ACH, ACH)])

    loads(0, 0)

    def bpair(t, carry):
        i0 = 2 * t
        loads(i0 + 1, 1)
        step(i0, 0)
        loads(i0 + 2, 0)
        step(i0 + 1, 1)
        return carry

    lax.fori_loop(0, (ANIT - 1) // 2, bpair, 0)
    step(ANIT - 1, 0)


@functools.partial(
    pl.kernel,
    out_type=jax.ShapeDtypeStruct((NCORES, N, HID), jnp.float32),
    mesh=_mesh,
    scratch_types=[
        pltpu.VMEM((PT + 16,), jnp.int32),
        pltpu.VMEM((K,), jnp.int32),
        pltpu.VMEM((K,), jnp.int32),
        pltpu.VMEM((K,), jnp.int32),
        pltpu.VMEM((K,), jnp.int32),
        pltpu.VMEM((2 * 16,), jnp.int32),
        pltpu.VMEM((2 * 16,), jnp.int32),
        pltpu.VMEM((2, K, G4), jnp.float32),
        pltpu.VMEM((2, K, SRCW), jnp.float32),
        pltpu.VMEM((2, K, G4), jnp.float32),
        pltpu.VMEM((K, HID), jnp.float32),
        pltpu.VMEM_SHARED((N, HID), jnp.float32),
        pltpu.SemaphoreType.DMA,
        pltpu.SemaphoreType.DMA,
        pltpu.SemaphoreType.DMA,
        pltpu.SemaphoreType.DMA,
        pltpu.SemaphoreType.DMA,
        pltpu.SemaphoreType.DMA,
    ],
)
def _paths(tdst_hbm, tsrc_hbm, tattr_hbm, packed_hbm, dsts_hbm, zeros_hbm,
           out_hbm,
           sa_loc, dstb0, dstb1, dstb2, dstb3, srcb, attrb, bufA, bufS, bufB,
           h_out, acc, sem0, sem1, semd0, semd1, semd2, semd3):
    sid = lax.axis_index("s")
    cid = lax.axis_index("c")
    wid = sid * NCORES + cid
    sems = (sem0, sem1)
    semd = (semd0, semd1, semd2, semd3)
    dstb = (dstb0, dstb1, dstb2, dstb3)

    pltpu.sync_copy(packed_hbm.at[pl.ds(wid * PT, PT)],
                    sa_loc.at[pl.ds(0, PT)])
    pltpu.sync_copy(zeros_hbm.at[pl.ds(0, NTROW)],
                    acc.at[pl.ds(sid * NTROW, NTROW)])

    @pl.when(sid == NSUB - 1)
    def _zero_tail():
        pltpu.sync_copy(zeros_hbm.at[pl.ds(0, NTAIL)],
                        acc.at[pl.ds(NTROW * NSUB, NTAIL)])

    plsc.subcore_barrier()

    def idx_load(g, d):
        pltpu.async_copy(dsts_hbm.at[pl.ds(wid * PT + g * K, K)],
                         dstb[d], semd[d])

    def start(g, d, b):
        pltpu.make_async_copy(dsts_hbm.at[pl.ds(0, K)], dstb[d],
                              semd[d]).wait()
        w16 = sa_loc[pl.ds(g * K, 16)]
        srcb[pl.ds(b * 16, 16)] = jnp.bitwise_and(w16, SMASK)
        attrb[pl.ds(b * 16, 16)] = jnp.right_shift(w16, 14)
        pltpu.async_copy(tdst_hbm.at[dstb[d]], bufA.at[b], sems[b])
        pltpu.async_copy(tsrc_hbm.at[srcb.at[pl.ds(b * 16, K)]],
                         bufS.at[b], sems[b])
        pltpu.async_copy(tattr_hbm.at[attrb.at[pl.ds(b * 16, K)]],
                         bufB.at[b], sems[b])

    def drain(b):
        pltpu.make_async_copy(tdst_hbm.at[pl.ds(0, K)], bufA.at[b],
                              sems[b]).wait()
        pltpu.make_async_copy(tsrc_hbm.at[pl.ds(0, K)], bufS.at[b],
                              sems[b]).wait()
        pltpu.make_async_copy(tattr_hbm.at[pl.ds(0, K)], bufB.at[b],
                              sems[b]).wait()

    def compute(g, d, b):
        def pbody(j, carry):
            for q in range(HID // 16):
                qo = q * 16
                gi = (bufA[b, j, pl.ds(qo, 16)]
                      + bufS[b, j, pl.ds(qo, 16)]
                      + bufB[b, j, pl.ds(qo, 16)])
                gf = (bufA[b, j, pl.ds(HID + qo, 16)]
                      + bufS[b, j, pl.ds(HID + qo, 16)]
                      + bufB[b, j, pl.ds(HID + qo, 16)])
                gg = (bufA[b, j, pl.ds(2 * HID + qo, 16)]
                      + bufS[b, j, pl.ds(2 * HID + qo, 16)]
                      + bufB[b, j, pl.ds(2 * HID + qo, 16)])
                go = (bufA[b, j, pl.ds(3 * HID + qo, 16)]
                      + bufS[b, j, pl.ds(3 * HID + qo, 16)]
                      + bufB[b, j, pl.ds(3 * HID + qo, 16)])
                c0q = bufS[b, j, pl.ds(G4 + qo, 16)]
                c1 = _sig(gf) * c0q + _sig(gi) * _th(gg)
                h_out[j, pl.ds(qo, 16)] = _sig(go) * _th(c1)
            return carry

        lax.fori_loop(0, K, pbody, 0)
        pltpu.sync_copy(h_out, acc.at[dstb[d]], add=True)

    for d in range(4):
        idx_load(d, d)
    start(0, 0, 0)

    def quad(t, carry):
        g = 4 * t
        start(g + 1, 1, 1)
        drain(0)
        compute(g, 0, 0)
        idx_load(g + 4, 0)
        start(g + 2, 2, 0)
        drain(1)
        compute(g + 1, 1, 1)
        idx_load(g + 5, 1)
        start(g + 3, 3, 1)
        drain(0)
        compute(g + 2, 2, 0)
        idx_load(g + 6, 2)
        start(g + 4, 0, 0)
        drain(1)
        compute(g + 3, 3, 1)
        idx_load(g + 7, 3)
        return carry

    NQ = (NCHUNK - 8) // 4 + 1
    lax.fori_loop(0, NQ, quad, 0)

    gE = NQ * 4
    start(gE + 1, (gE + 1) % 4, 1)
    drain(0)
    compute(gE, gE % 4, 0)
    idx_load(gE + 4, (gE + 4) % 4)
    start(gE + 2, (gE + 2) % 4, 0)
    drain(1)
    compute(gE + 1, (gE + 1) % 4, 1)
    idx_load(gE + 5, (gE + 5) % 4)
    start(gE + 3, (gE + 3) % 4, 1)
    drain(0)
    compute(gE + 2, (gE + 2) % 4, 0)
    start(gE + 4, (gE + 4) % 4, 0)
    drain(1)
    compute(gE + 3, (gE + 3) % 4, 1)
    start(gE + 5, (gE + 5) % 4, 1)
    drain(0)
    compute(gE + 4, (gE + 4) % 4, 0)
    drain(1)
    compute(gE + 5, (gE + 5) % 4, 1)

    plsc.subcore_barrier()
    pltpu.sync_copy(acc.at[pl.ds(sid * NTROW, NTROW)],
                    out_hbm.at[cid, pl.ds(sid * NTROW, NTROW)])

    @pl.when(sid == NSUB - 1)
    def _write_tail():
        pltpu.sync_copy(acc.at[pl.ds(NTROW * NSUB, NTAIL)],
                        out_hbm.at[cid, pl.ds(NTROW * NSUB, NTAIL)])


def _tables_tc(wpad, wihxT, whhT, bias2):
    BR = 1024

    def body(w_ref, wx_ref, wh_ref, b_ref, tdst_ref, tsrc_ref):
        a = jnp.dot(w_ref[...], wx_ref[...],
                    preferred_element_type=jnp.float32)
        tdst_ref[...] = a
        g0 = a + b_ref[...]
        c0 = _sig(g0[:, :HID]) * jnp.tanh(g0[:, 2 * HID:3 * HID])
        h0 = _sig(g0[:, 3 * HID:]) * jnp.tanh(c0)
        cc = jnp.dot(h0, wh_ref[...], preferred_element_type=jnp.float32)
        tsrc_ref[:, :G4] = cc
        tsrc_ref[:, G4:] = c0

    return pl.pallas_call(
        body,
        grid=(NPAD // BR,),
        in_specs=[
            pl.BlockSpec((BR, HID), lambda i: (i, 0)),
            pl.BlockSpec((HID, G4), lambda i: (0, 0)),
            pl.BlockSpec((HID, G4), lambda i: (0, 0)),
            pl.BlockSpec((1, G4), lambda i: (0, 0)),
        ],
        out_specs=[
            pl.BlockSpec((BR, G4), lambda i: (i, 0)),
            pl.BlockSpec((BR, SRCW), lambda i: (i, 0)),
        ],
        out_shape=[
            jax.ShapeDtypeStruct((NPAD, G4), jnp.float32),
            jax.ShapeDtypeStruct((NPAD, SRCW), jnp.float32),
        ],
    )(wpad, wihxT, whhT, bias2)


def _tattr_tc(embe_pad, wiheT, bias2):
    def body(e_ref, wx_ref, b_ref, o_ref):
        o_ref[...] = jnp.dot(e_ref[...], wx_ref[...],
                             preferred_element_type=jnp.float32) + b_ref[...]

    return pl.pallas_call(
        body,
        out_shape=jax.ShapeDtypeStruct((TAROWS, G4), jnp.float32),
    )(embe_pad, wiheT, bias2)


def _post_tc(part2, batch2, bng, bnb, w1T, b1, g1, e1, w2T, b2, g2, e2,
             l1T, lb1, l2Tp, lb2p):
    def bn(h, g, bb):
        m = jnp.mean(h, axis=0, keepdims=True)
        v = jnp.mean((h - m) ** 2, axis=0, keepdims=True)
        return (h - m) / jnp.sqrt(v + EPS) * g + bb

    def body(part_ref, batch_ref, bng_ref, bnb_ref, w1_ref, b1_ref, g1_ref,
             e1_ref, w2_ref, b2_ref, g2_ref, e2_ref, l1_ref, lb1_ref,
             l2_ref, lb2_ref, out_ref):
        h = part_ref[:N, :] + part_ref[N:, :]
        h = bn(h, bng_ref[...], bnb_ref[...])
        h = jnp.dot(h, w1_ref[...], preferred_element_type=jnp.float32)
        h = jnp.maximum(bn(h + b1_ref[...], g1_ref[...], e1_ref[...]), 0.0)
        h = jnp.dot(h, w2_ref[...], preferred_element_type=jnp.float32)
        h = jnp.maximum(bn(h + b2_ref[...], g2_ref[...], e2_ref[...]), 0.0)
        seg = lax.broadcasted_iota(jnp.int32, (NG, N), 0)
        oh = (seg == batch_ref[...]).astype(jnp.float32)
        pooled = jnp.dot(oh, h, preferred_element_type=jnp.float32)
        o1 = jnp.maximum(
            jnp.dot(pooled, l1_ref[...],
                    preferred_element_type=jnp.float32) + lb1_ref[...], 0.0)
        out_ref[...] = jnp.dot(
            o1, l2_ref[...], preferred_element_type=jnp.float32) + lb2_ref[...]

    return pl.pallas_call(
        body,
        out_shape=jax.ShapeDtypeStruct((NG, HID), jnp.float32),
    )(part2, batch2, bng, bnb, w1T, b1, g1, e1, w2T, b2, g2, e2,
      l1T, lb1, l2Tp, lb2p)


def kernel(x, edge_attr, path_2, edge_indices_2, batch, emb_v, emb_e,
           W_ih, W_hh, b_ih, b_hh, bn_g, bn_b, mlp_W1, mlp_b1, bn1_g, bn1_b,
           mlp_W2, mlp_b2, bn2_g, bn2_b, lin1_W, lin1_b, lin2_W, lin2_b):
    f32 = jnp.float32
    x_pad = jnp.pad(x, (0, NPAD - N))
    srcs = path_2[:, 0].reshape(P)
    dsts = path_2[:, 1].reshape(P)
    es = edge_indices_2[:, 0].reshape(P)
    wihxT = W_ih[:, :HID].T.astype(f32)
    wiheT = W_ih[:, HID:].T.astype(f32)
    whhT = W_hh.T.astype(f32)
    bias2 = (b_ih + b_hh).reshape(1, G4)
    embe_pad = jnp.pad(emb_e, ((0, TAROWS - VE), (0, 0)))
    zeros = jnp.zeros((NTROW, HID), f32)

    wpad, packed = _gather_stage(x_pad, emb_v, es, srcs, edge_attr)
    tdst, tsrc = _tables_tc(wpad, wihxT, whhT, bias2)
    tattr = _tattr_tc(embe_pad, wiheT, bias2)
    part = _paths(tdst, tsrc, tattr, packed, dsts, zeros)
    out = _post_tc(
        part.reshape(NCORES * N, HID), batch.reshape(1, N),
        bn_g.reshape(1, HID), bn_b.reshape(1, HID),
        mlp_W1.T, mlp_b1.reshape(1, HID),
        bn1_g.reshape(1, HID), bn1_b.reshape(1, HID),
        mlp_W2.T, mlp_b2.reshape(1, HID),
        bn2_g.reshape(1, HID), bn2_b.reshape(1, HID),
        lin1_W.T, lin1_b.reshape(1, HID),
        jnp.pad(lin2_W.T, ((0, 0), (0, HID - NCLS))),
        jnp.pad(lin2_b, (0, HID - NCLS)).reshape(1, HID),
    )
    return out[:, :NCLS]
